# baseline (speedup 1.0000x reference)
import jax
import jax.numpy as jnp
from jax.experimental import pallas as pl
from jax.experimental.pallas import tpu as pltpu

_BK = 512
_SPLIT = 3
_DN = (((1,), (1,)), ((), ()))


def _row_splits(nrows):
    base, rem = divmod(nrows, _SPLIT)
    offs, o = [], 0
    for i in range(_SPLIT):
        sz = base + (1 if i < rem else 0)
        sz = (sz + 7) // 8 * 8 if i < _SPLIT - 1 else nrows - o
        offs.append((o, sz))
        o += sz
    return offs


def _mlp_kernel(x_hbm, w1_hbm, b1_hbm, w2_hbm, b2_hbm, w3_hbm, b3_hbm,
                o_ref, xv, wbuf, h1, h2, w3v, b1v, b2v, b3v,
                sem_w, sem_x, sem_w3, sem_b):
    k = w1_hbm.shape[0] // _BK
    n = 2 * k

    def w_copies(t):
        w_hbm = w1_hbm if t < k else w2_hbm
        r0 = (t % k) * _BK
        return [pltpu.make_async_copy(
                    w_hbm.at[pl.ds(r0 + o, sz), :],
                    wbuf.at[t, pl.ds(o, sz), :],
                    sem_w.at[t, i])
                for i, (o, sz) in enumerate(_row_splits(_BK))]

    cp_x = [pltpu.make_async_copy(x_hbm.at[pl.ds(i * 128, 128), :],
                                  xv.at[pl.ds(i * 128, 128), :], sem_x.at[i])
            for i in range(2)]
    cp_b1 = pltpu.make_async_copy(b1_hbm, b1v, sem_b.at[0])
    cp_b2 = pltpu.make_async_copy(b2_hbm, b2v, sem_b.at[1])
    cp_b3 = pltpu.make_async_copy(b3_hbm, b3v, sem_b.at[2])
    cp_w3 = pltpu.make_async_copy(w3_hbm, w3v, sem_w3)

    for c in cp_x:
        c.start()
    cp_b1.start()
    cp_b2.start()
    for c in w_copies(0):
        c.start()

    for c in cp_x:
        c.wait()
    cp_b1.wait()
    cp_b2.wait()
    for c in w_copies(1):
        c.start()

    for t in range(n):
        for c in w_copies(t):
            c.wait()
        if t + 2 < n:
            for c in w_copies(t + 2):
                c.start()
        elif t + 2 == n:
            cp_w3.start()
            cp_b3.start()
        if t < k:
            h = jax.lax.dot_general(xv[...], wbuf[t], _DN,
                                    preferred_element_type=jnp.float32)
            h1[:, pl.ds(t * _BK, _BK)] = jnp.maximum(
                h + b1v[:, pl.ds(t * _BK, _BK)], 0.0)
        else:
            j = t - k
            h = jax.lax.dot_general(h1[...], wbuf[t], _DN,
                                    preferred_element_type=jnp.float32)
            h2[:, pl.ds(j * _BK, _BK)] = jnp.maximum(
                h + b2v[:, pl.ds(j * _BK, _BK)], 0.0)

    cp_w3.wait()
    cp_b3.wait()
    o = jax.lax.dot_general(h2[...], w3v[...], _DN,
                            preferred_element_type=jnp.float32)
    o_ref[...] = o + b3v[...]


def kernel(x, W1, b1, W2, b2, W3, b3, t):
    del t
    B, D_IN = x.shape
    D_H = W1.shape[0]
    D_OUT = W3.shape[0]
    n = 2 * D_H // _BK
    return pl.pallas_call(
        _mlp_kernel,
        in_specs=[pl.BlockSpec(memory_space=pl.ANY)] * 7,
        out_specs=pl.BlockSpec((B, D_OUT), lambda: (0, 0)),
        out_shape=jax.ShapeDtypeStruct((B, D_OUT), jnp.float32),
        scratch_shapes=[
            pltpu.VMEM((B, D_IN), jnp.float32),
            pltpu.VMEM((n, _BK, D_IN), jnp.float32),
            pltpu.VMEM((B, D_H), jnp.float32),
            pltpu.VMEM((B, D_H), jnp.float32),
            pltpu.VMEM((D_OUT, D_H), jnp.float32),
            pltpu.VMEM((1, D_H), jnp.float32),
            pltpu.VMEM((1, D_H), jnp.float32),
            pltpu.VMEM((1, D_OUT), jnp.float32),
            pltpu.SemaphoreType.DMA((n, _SPLIT)),
            pltpu.SemaphoreType.DMA((2,)),
            pltpu.SemaphoreType.DMA,
            pltpu.SemaphoreType.DMA((3,)),
        ],
    )(x, W1, b1.reshape(1, -1), W2, b2.reshape(1, -1), W3, b3.reshape(1, -1))


# speedup vs baseline: 1.0192x; 1.0192x over previous
import jax
import jax.numpy as jnp
from jax.experimental import pallas as pl
from jax.experimental.pallas import tpu as pltpu

_BK = 512
_SPLIT = 3
_DN = (((1,), (1,)), ((), ()))


def _row_splits(nrows):
    base, rem = divmod(nrows, _SPLIT)
    offs, o = [], 0
    for i in range(_SPLIT):
        sz = base + (1 if i < rem else 0)
        sz = (sz + 7) // 8 * 8 if i < _SPLIT - 1 else nrows - o
        offs.append((o, sz))
        o += sz
    return offs


def _mlp_kernel(x_hbm, w1_hbm, b1_hbm, w2_hbm, b2_hbm, w3_hbm, b3_hbm,
                o_ref, xv, wbuf, h1, h2, w3v, b1v, b2v, b3v,
                sem_w, sem_x, sem_w3, sem_b):
    k = w1_hbm.shape[0] // _BK
    n = 2 * k

    def w_copies(t):
        w_hbm = w1_hbm if t < k else w2_hbm
        r0 = (t % k) * _BK
        return [pltpu.make_async_copy(
                    w_hbm.at[pl.ds(r0 + o, sz), :],
                    wbuf.at[t, pl.ds(o, sz), :],
                    sem_w.at[t])
                for o, sz in _row_splits(_BK)]

    def w_wait(t):
        pltpu.make_async_copy(
            w1_hbm.at[pl.ds(0, _BK), :], wbuf.at[t], sem_w.at[t]).wait()

    cp_x = [pltpu.make_async_copy(x_hbm.at[pl.ds(i * 128, 128), :],
                                  xv.at[pl.ds(i * 128, 128), :], sem_x.at[i])
            for i in range(2)]
    cp_b1 = pltpu.make_async_copy(b1_hbm, b1v, sem_b.at[0])
    cp_b2 = pltpu.make_async_copy(b2_hbm, b2v, sem_b.at[1])
    cp_b3 = pltpu.make_async_copy(b3_hbm, b3v, sem_b.at[2])
    cp_w3 = pltpu.make_async_copy(w3_hbm, w3v, sem_w3)

    for c in cp_x:
        c.start()
    cp_b1.start()
    cp_b2.start()
    for c in w_copies(0):
        c.start()

    for c in cp_x:
        c.wait()
    cp_b1.wait()
    cp_b2.wait()
    for c in w_copies(1):
        c.start()

    for t in range(n):
        w_wait(t)
        if t + 2 < n:
            for c in w_copies(t + 2):
                c.start()
        elif t + 2 == n:
            cp_w3.start()
            cp_b3.start()
        if t < k:
            h = jax.lax.dot_general(xv[...], wbuf[t], _DN,
                                    preferred_element_type=jnp.float32)
            h1[:, pl.ds(t * _BK, _BK)] = jnp.maximum(
                h + b1v[:, pl.ds(t * _BK, _BK)], 0.0)
        else:
            j = t - k
            h = jax.lax.dot_general(h1[...], wbuf[t], _DN,
                                    preferred_element_type=jnp.float32)
            h2[:, pl.ds(j * _BK, _BK)] = jnp.maximum(
                h + b2v[:, pl.ds(j * _BK, _BK)], 0.0)

    cp_w3.wait()
    cp_b3.wait()
    o = jax.lax.dot_general(h2[...], w3v[...], _DN,
                            preferred_element_type=jnp.float32)
    o_ref[...] = o + b3v[...]


def kernel(x, W1, b1, W2, b2, W3, b3, t):
    del t
    B, D_IN = x.shape
    D_H = W1.shape[0]
    D_OUT = W3.shape[0]
    n = 2 * D_H // _BK
    return pl.pallas_call(
        _mlp_kernel,
        in_specs=[pl.BlockSpec(memory_space=pl.ANY)] * 7,
        out_specs=pl.BlockSpec((B, D_OUT), lambda: (0, 0)),
        out_shape=jax.ShapeDtypeStruct((B, D_OUT), jnp.float32),
        scratch_shapes=[
            pltpu.VMEM((B, D_IN), jnp.float32),
            pltpu.VMEM((n, _BK, D_IN), jnp.float32),
            pltpu.VMEM((B, D_H), jnp.float32),
            pltpu.VMEM((B, D_H), jnp.float32),
            pltpu.VMEM((D_OUT, D_H), jnp.float32),
            pltpu.VMEM((1, D_H), jnp.float32),
            pltpu.VMEM((1, D_H), jnp.float32),
            pltpu.VMEM((1, D_OUT), jnp.float32),
            pltpu.SemaphoreType.DMA((n,)),
            pltpu.SemaphoreType.DMA((2,)),
            pltpu.SemaphoreType.DMA,
            pltpu.SemaphoreType.DMA((3,)),
        ],
    )(x, W1, b1.reshape(1, -1), W2, b2.reshape(1, -1), W3, b3.reshape(1, -1))
